# baseline (speedup 1.0000x reference)
import functools

import jax
import jax.numpy as jnp
from jax.experimental import pallas as pl
from jax.experimental.pallas import tpu as pltpu

B = 16384
D_IN = 17
D_OUT = 6
N_CLUSTERS = 4
H_EXP = 8


def _fused_kernel(xt_ref, gW1_ref, gb1_ref, gW2_ref, gb2_ref, gW3_ref, gb3_ref,
                  eW1_ref, eb1_ref, eW2_ref, eb2_ref, eW3_ref, eb3_ref,
                  pred_ref, logits_ref):
    f32 = jnp.float32
    xT = xt_ref[...]

    h = jnp.maximum(jnp.dot(gW1_ref[...].T, xT, preferred_element_type=f32)
                    + gb1_ref[...].T, 0.0)
    h = jnp.maximum(jnp.dot(gW2_ref[...].T, h, preferred_element_type=f32)
                    + gb2_ref[...].T, 0.0)
    logits = (jnp.dot(gW3_ref[...].T, h, preferred_element_type=f32)
              + gb3_ref[...].T)
    logits_ref[...] = logits

    m = jnp.max(logits, axis=0, keepdims=True)
    iota4 = jax.lax.broadcasted_iota(jnp.int32, (N_CLUSTERS, B), 0)
    sel = jnp.min(jnp.where(logits == m, iota4, N_CLUSTERS),
                  axis=0, keepdims=True)

    e1t = jnp.concatenate([eW1_ref[e].T for e in range(N_CLUSTERS)], axis=0)
    b1c = jnp.concatenate([eb1_ref[e:e + 1, :].T for e in range(N_CLUSTERS)], axis=0)
    h1 = jnp.maximum(jnp.dot(e1t, xT, preferred_element_type=f32) + b1c, 0.0)

    z8 = jnp.zeros((H_EXP, H_EXP), f32)
    e2rows = []
    for e in range(N_CLUSTERS):
        row = [eW2_ref[e].T if j == e else z8 for j in range(N_CLUSTERS)]
        e2rows.append(jnp.concatenate(row, axis=1))
    e2bd = jnp.concatenate(e2rows, axis=0)
    b2c = jnp.concatenate([eb2_ref[e:e + 1, :].T for e in range(N_CLUSTERS)], axis=0)
    h2 = jnp.maximum(jnp.dot(e2bd, h1, preferred_element_type=f32) + b2c, 0.0)

    group = jax.lax.broadcasted_iota(jnp.int32, (N_CLUSTERS * H_EXP, B), 0) // H_EXP
    h2m = jnp.where(group == sel, h2, 0.0)
    e3t = jnp.concatenate([eW3_ref[e].T for e in range(N_CLUSTERS)], axis=1)
    onehot = (iota4 == sel).astype(f32)
    pred_ref[...] = (jnp.dot(e3t, h2m, preferred_element_type=f32)
                     + jnp.dot(eb3_ref[...].T, onehot, preferred_element_type=f32))


@functools.partial(jax.jit, static_argnames=())
def kernel(x, gW1, gb1, gW2, gb2, gW3, gb3, eW1, eb1, eW2, eb2, eW3, eb3):
    xt = x.T

    gb1r, gb2r, gb3r = gb1.reshape(1, -1), gb2.reshape(1, -1), gb3.reshape(1, -1)
    ins = (xt, gW1, gb1r, gW2, gb2r, gW3, gb3r, eW1, eb1, eW2, eb2, eW3, eb3)
    full = lambda a: pl.BlockSpec(a.shape, lambda: (0,) * a.ndim)

    predT, logitsT = pl.pallas_call(
        _fused_kernel,
        in_specs=[full(a) for a in ins],
        out_specs=[
            pl.BlockSpec((D_OUT, B), lambda: (0, 0)),
            pl.BlockSpec((N_CLUSTERS, B), lambda: (0, 0)),
        ],
        out_shape=[
            jax.ShapeDtypeStruct((D_OUT, B), jnp.float32),
            jax.ShapeDtypeStruct((N_CLUSTERS, B), jnp.float32),
        ],
    )(*ins)
    return predT.T, logitsT.T
